# Initial kernel scaffold; baseline (speedup 1.0000x reference)
import functools

import jax
import jax.numpy as jnp
from jax import lax
from jax.experimental import pallas as pl
from jax.experimental.pallas import tpu as pltpu
from jax.experimental.pallas import tpu_sc as plsc

N = 10000
E = 320000
NC, NS = 2, 16
NW = NC * NS
EW = E // NW
CH = 80
NCH = EW // CH
RPS = N // NS
HW = 16

_MESH = plsc.VectorSubcoreMesh(core_axis_name="c", subcore_axis_name="s")


def _zero_vmem(ref, nrows, d):
    zv = jnp.zeros((16,), jnp.float32)

    def row(r, _):
        def col(j, _):
            ref[r, pl.ds(j * 16, 16)] = zv
            return 0

        return lax.fori_loop(0, d // 16, col, 0)

    lax.fori_loop(0, nrows, row, 0)


@functools.partial(
    pl.kernel,
    out_type=jax.ShapeDtypeStruct((NC, 4, N, HW), jnp.float32),
    mesh=_MESH,
    scratch_types=[
        pltpu.VMEM((NCH, CH), jnp.int32),
        pltpu.VMEM((CH, HW), jnp.float32),
        pltpu.VMEM((125, HW), jnp.float32),
        pltpu.VMEM_SHARED((4, N, HW), jnp.float32),
    ],
)
def _sc_degrees(idx_hbm, out_hbm, idx_v, ones_v, z_v, hist_sh):
    c = lax.axis_index("c")
    s = lax.axis_index("s")
    w = c * NS + s

    one_row = jnp.where(lax.iota(jnp.int32, 16) == 0, 1.0, 0.0).astype(jnp.float32)

    def setrow(r, _):
        ones_v[r, :] = one_row
        return 0

    lax.fori_loop(0, CH, setrow, 0)
    _zero_vmem(z_v, 125, HW)

    for k in range(4):
        for i in range(RPS // 125):
            pltpu.sync_copy(z_v, hist_sh.at[k, pl.ds(s * RPS + i * 125, 125)])
    plsc.subcore_barrier()

    for k in range(4):
        pltpu.sync_copy(idx_hbm.at[k, w], idx_v)

        def chunk(j, _):
            pltpu.sync_copy(ones_v, hist_sh.at[k, idx_v.at[j]], add=True)
            return 0

        lax.fori_loop(0, NCH, chunk, 0)
    plsc.subcore_barrier()

    for k in range(4):
        pltpu.sync_copy(
            hist_sh.at[k, pl.ds(s * RPS, RPS)],
            out_hbm.at[c, k, pl.ds(s * RPS, RPS)],
        )


def _make_sc_aggregate(d):
    @functools.partial(
        pl.kernel,
        out_type=jax.ShapeDtypeStruct((NC, N, d), jnp.float32),
        mesh=_MESH,
        scratch_types=[
            pltpu.VMEM((NCH, CH), jnp.int32),
            pltpu.VMEM((NCH, CH), jnp.int32),
            pltpu.VMEM((CH, d), jnp.float32),
            pltpu.VMEM((125, d), jnp.float32),
            pltpu.VMEM_SHARED((N, d), jnp.float32),
            pltpu.SemaphoreType.DMA,
        ],
    )
    def agg(h_hbm, src_hbm, dst_hbm, out_hbm, src_v, dst_v, rows_v, z_v, acc_sh, sem):
        c = lax.axis_index("c")
        s = lax.axis_index("s")
        w = c * NS + s

        _zero_vmem(z_v, 125, d)
        for i in range(RPS // 125):
            pltpu.sync_copy(z_v, acc_sh.at[pl.ds(s * RPS + i * 125, 125)])
        plsc.subcore_barrier()

        pltpu.sync_copy(src_hbm.at[w], src_v)
        pltpu.sync_copy(dst_hbm.at[w], dst_v)

        def chunk(j, _):
            pltpu.async_copy(h_hbm.at[src_v.at[j]], rows_v, sem).wait()
            pltpu.sync_copy(rows_v, acc_sh.at[dst_v.at[j]], add=True)
            return 0

        lax.fori_loop(0, NCH, chunk, 0)
        plsc.subcore_barrier()

        pltpu.sync_copy(
            acc_sh.at[pl.ds(s * RPS, RPS)],
            out_hbm.at[c, pl.ds(s * RPS, RPS)],
        )

    return agg


_sc_aggregate_128 = _make_sc_aggregate(128)
_sc_aggregate_64 = _make_sc_aggregate(64)


def _norm_col(degp_ref):
    deg = degp_ref[0, :, 0:1] + degp_ref[1, :, 0:1]
    return lax.rsqrt(jnp.maximum(deg, 1.0))


def _scale_body(x_ref, degp_ref, o_ref):
    o_ref[...] = x_ref[...] * _norm_col(degp_ref)


def _mm_body(aggp_ref, d1_ref, d2_ref, w1_ref, b1_ref, w2_ref, o_ref):
    a = (aggp_ref[0] + aggp_ref[1]) * _norm_col(d1_ref)
    m = jnp.dot(a, w1_ref[...], preferred_element_type=jnp.float32) + b1_ref[...]
    m = m * _norm_col(d2_ref)
    o_ref[...] = jnp.dot(m, w2_ref[...], preferred_element_type=jnp.float32)


def _final_body(aggp_ref, d2_ref, b2_ref, o_ref):
    a = (aggp_ref[0] + aggp_ref[1]) * _norm_col(d2_ref)
    o_ref[...] = jax.nn.sigmoid(a + b2_ref[...])


_BR = 2000
_tc_scale = pl.pallas_call(
    _scale_body,
    grid=(N // _BR,),
    in_specs=[
        pl.BlockSpec((_BR, 128), lambda i: (i, 0)),
        pl.BlockSpec((NC, _BR, HW), lambda i: (0, i, 0)),
    ],
    out_specs=pl.BlockSpec((_BR, 128), lambda i: (i, 0)),
    out_shape=jax.ShapeDtypeStruct((N, 128), jnp.float32),
)

_BM = 1000
_tc_matmul = pl.pallas_call(
    _mm_body,
    grid=(N // _BM,),
    in_specs=[
        pl.BlockSpec((NC, _BM, 128), lambda i: (0, i, 0)),
        pl.BlockSpec((NC, _BM, HW), lambda i: (0, i, 0)),
        pl.BlockSpec((NC, _BM, HW), lambda i: (0, i, 0)),
        pl.BlockSpec((128, 1000), lambda i: (0, 0)),
        pl.BlockSpec((1, 1000), lambda i: (0, 0)),
        pl.BlockSpec((1000, 64), lambda i: (0, 0)),
    ],
    out_specs=pl.BlockSpec((_BM, 64), lambda i: (i, 0)),
    out_shape=jax.ShapeDtypeStruct((N, 64), jnp.float32),
)

_tc_final = pl.pallas_call(
    _final_body,
    grid=(N // _BR,),
    in_specs=[
        pl.BlockSpec((NC, _BR, 64), lambda i: (0, i, 0)),
        pl.BlockSpec((NC, _BR, HW), lambda i: (0, i, 0)),
        pl.BlockSpec((1, 64), lambda i: (0, 0)),
    ],
    out_specs=pl.BlockSpec((_BR, 64), lambda i: (i, 0)),
    out_shape=jax.ShapeDtypeStruct((N, 64), jnp.float32),
)


def kernel(x, edge_index1, edge_index2, W1, b1, W2, b2):
    src1 = edge_index1[0].astype(jnp.int32).reshape(NW, NCH, CH)
    dst1 = edge_index1[1].astype(jnp.int32).reshape(NW, NCH, CH)
    src2 = edge_index2[0].astype(jnp.int32).reshape(NW, NCH, CH)
    dst2 = edge_index2[1].astype(jnp.int32).reshape(NW, NCH, CH)

    hist = _sc_degrees(jnp.stack([src1, dst1, src2, dst2]))
    outdeg1 = hist[:, 0]
    indeg1 = hist[:, 1]
    outdeg2 = hist[:, 2]
    indeg2 = hist[:, 3]

    h1 = _tc_scale(x, outdeg1)
    agg1 = _sc_aggregate_128(h1, src1, dst1)
    t = _tc_matmul(agg1, indeg1, outdeg2, W1, b1.reshape(1, 1000), W2)
    agg2 = _sc_aggregate_64(t, src2, dst2)
    return _tc_final(agg2, indeg2, b2.reshape(1, 64))


# speedup vs baseline: 10.7460x; 10.7460x over previous
import functools

import jax
import jax.numpy as jnp
from jax import lax
from jax.experimental import pallas as pl
from jax.experimental.pallas import tpu as pltpu
from jax.experimental.pallas import tpu_sc as plsc

N = 10000
NP = 10240
E = 320000
NC, NS = 2, 16
NW = NC * NS
EW = E // NW
CH = 80
IB = 25
NB = EW // (IB * CH)
ZR = 32
HW = 16

_MESH = plsc.VectorSubcoreMesh(core_axis_name="c", subcore_axis_name="s")


def _zero_vmem(ref, nrows, d):
    zv = jnp.zeros((16,), jnp.float32)

    def row(r, _):
        def col(j, _):
            ref[r, pl.ds(j * 16, 16)] = zv
            return 0

        return lax.fori_loop(0, d // 16, col, 0)

    lax.fori_loop(0, nrows, row, 0)


def _make_sc_aggregate(d, nrows, nblocks):
    rps = nrows // NS

    @functools.partial(
        pl.kernel,
        out_type=jax.ShapeDtypeStruct((NC, nrows, d), jnp.float32),
        mesh=_MESH,
        scratch_types=[
            pltpu.VMEM((IB, CH), jnp.int32),
            pltpu.VMEM((IB, CH), jnp.int32),
            pltpu.VMEM((CH, d), jnp.float32),
            pltpu.VMEM((ZR, d), jnp.float32),
            pltpu.VMEM_SHARED((nrows, d), jnp.float32),
            pltpu.SemaphoreType.DMA,
        ],
    )
    def agg(h_hbm, src_hbm, dst_hbm, out_hbm, src_v, dst_v, rows_v, z_v, acc_sh, sem):
        c = lax.axis_index("c")
        s = lax.axis_index("s")
        w = c * NS + s

        _zero_vmem(z_v, ZR, d)
        for i in range(rps // ZR):
            pltpu.sync_copy(z_v, acc_sh.at[pl.ds(s * rps + i * ZR, ZR)])
        plsc.subcore_barrier()

        def block(b, _):
            pltpu.sync_copy(src_hbm.at[w, b], src_v)
            pltpu.sync_copy(dst_hbm.at[w, b], dst_v)

            def chunk(j, _):
                pltpu.async_copy(h_hbm.at[src_v.at[j]], rows_v, sem).wait()
                pltpu.sync_copy(rows_v, acc_sh.at[dst_v.at[j]], add=True)
                return 0

            lax.fori_loop(0, IB, chunk, 0)
            return 0

        lax.fori_loop(0, nblocks, block, 0)
        plsc.subcore_barrier()

        pltpu.sync_copy(
            acc_sh.at[pl.ds(s * rps, rps)],
            out_hbm.at[c, pl.ds(s * rps, rps)],
        )

    return agg


_sc_aggregate_128 = _make_sc_aggregate(128, NP, NB)


_RPS = NP // NS


@functools.partial(
    pl.kernel,
    out_type=jax.ShapeDtypeStruct((NC, NP, 128), jnp.float32),
    mesh=_MESH,
    scratch_types=[
        pltpu.VMEM((IB, CH), jnp.int32),
        pltpu.VMEM((CH, 128), jnp.float32),
        pltpu.VMEM((ZR, 128), jnp.float32),
        pltpu.VMEM_SHARED((NP, 128), jnp.float32),
    ],
)
def _sc_histogram(ones_hbm, idx_hbm, out_hbm, idx_v, ones_v, z_v, acc_sh):
    c = lax.axis_index("c")
    s = lax.axis_index("s")
    w = c * NS + s

    _zero_vmem(z_v, ZR, 128)
    for i in range(_RPS // ZR):
        pltpu.sync_copy(z_v, acc_sh.at[pl.ds(s * _RPS + i * ZR, ZR)])
    plsc.subcore_barrier()

    for k in range(4):
        pltpu.sync_copy(ones_hbm.at[k], ones_v)

        def block(b, _, _k=k):
            pltpu.sync_copy(idx_hbm.at[w, _k, b], idx_v)

            def chunk(j, _):
                pltpu.sync_copy(ones_v, acc_sh.at[idx_v.at[j]], add=True)
                return 0

            lax.fori_loop(0, IB, chunk, 0)
            return 0

        lax.fori_loop(0, NB, block, 0)
    plsc.subcore_barrier()

    pltpu.sync_copy(
        acc_sh.at[pl.ds(s * _RPS, _RPS)],
        out_hbm.at[c, pl.ds(s * _RPS, _RPS)],
    )


def _norm_col(degp_ref, k):
    deg = degp_ref[0, :, k:k + 1] + degp_ref[1, :, k:k + 1]
    return lax.rsqrt(jnp.maximum(deg, 1.0))


def _scale_body(x_ref, degp_ref, o_ref):
    o_ref[...] = x_ref[...] * _norm_col(degp_ref, 0)


def _mm_body(aggp_ref, deg_ref, w1_ref, b1_ref, w2_ref, o_ref):
    a = (aggp_ref[0] + aggp_ref[1]) * _norm_col(deg_ref, 1)
    m = jnp.dot(a, w1_ref[...], preferred_element_type=jnp.float32) + b1_ref[...]
    m = m * _norm_col(deg_ref, 2)
    o_ref[...] = jnp.dot(m, w2_ref[...], preferred_element_type=jnp.float32)


def _final_body(aggp_ref, deg_ref, b2_ref, o_ref):
    a = (aggp_ref[0, :, 0:64] + aggp_ref[1, :, 0:64]) * _norm_col(deg_ref, 3)
    o_ref[...] = jax.nn.sigmoid(a + b2_ref[...])


_BR = 2000
_tc_scale = pl.pallas_call(
    _scale_body,
    grid=(N // _BR,),
    in_specs=[
        pl.BlockSpec((_BR, 128), lambda i: (i, 0)),
        pl.BlockSpec((NC, _BR, HW), lambda i: (0, i, 0)),
    ],
    out_specs=pl.BlockSpec((_BR, 128), lambda i: (i, 0)),
    out_shape=jax.ShapeDtypeStruct((N, 128), jnp.float32),
)

_BM = 1000
_tc_matmul = pl.pallas_call(
    _mm_body,
    grid=(N // _BM,),
    in_specs=[
        pl.BlockSpec((NC, _BM, 128), lambda i: (0, i, 0)),
        pl.BlockSpec((NC, _BM, HW), lambda i: (0, i, 0)),
        pl.BlockSpec((128, 1000), lambda i: (0, 0)),
        pl.BlockSpec((1, 1000), lambda i: (0, 0)),
        pl.BlockSpec((1000, 128), lambda i: (0, 0)),
    ],
    out_specs=pl.BlockSpec((_BM, 128), lambda i: (i, 0)),
    out_shape=jax.ShapeDtypeStruct((N, 128), jnp.float32),
)

_tc_final = pl.pallas_call(
    _final_body,
    grid=(N // _BR,),
    in_specs=[
        pl.BlockSpec((NC, _BR, 128), lambda i: (0, i, 0)),
        pl.BlockSpec((NC, _BR, HW), lambda i: (0, i, 0)),
        pl.BlockSpec((1, 64), lambda i: (0, 0)),
    ],
    out_specs=pl.BlockSpec((_BR, 64), lambda i: (i, 0)),
    out_shape=jax.ShapeDtypeStruct((N, 64), jnp.float32),
)


def kernel(x, edge_index1, edge_index2, W1, b1, W2, b2):
    src1 = edge_index1[0].astype(jnp.int32).reshape(NW, NB, IB, CH)
    dst1 = edge_index1[1].astype(jnp.int32).reshape(NW, NB, IB, CH)
    src2 = edge_index2[0].astype(jnp.int32).reshape(NW, NB, IB, CH)
    dst2 = edge_index2[1].astype(jnp.int32).reshape(NW, NB, IB, CH)

    idxs = jnp.stack([src1, dst1, src2, dst2], axis=1)
    onehots = jnp.broadcast_to(
        jnp.eye(128, dtype=jnp.float32)[:4][:, None, :], (4, CH, 128)
    )
    hist = _sc_histogram(onehots, idxs)[:, :, :HW]

    h1 = _tc_scale(x, hist)
    agg1 = _sc_aggregate_128(h1, src1, dst1)
    W2p = jnp.pad(W2, ((0, 0), (0, 64)))
    t = _tc_matmul(agg1, hist, W1, b1.reshape(1, 1000), W2p)
    agg2 = _sc_aggregate_128(t, src2, dst2)
    return _tc_final(agg2, hist, b2.reshape(1, 64))


# speedup vs baseline: 12.4819x; 1.1615x over previous
import functools

import jax
import jax.numpy as jnp
from jax import lax
from jax.experimental import pallas as pl
from jax.experimental.pallas import tpu as pltpu
from jax.experimental.pallas import tpu_sc as plsc

N = 10000
NP = 10240
E = 320000
NC, NS = 2, 16
NW = NC * NS
EW = E // NW
CH = 80
IB = 25
NB = EW // (IB * CH)
ZR = 32
HW = 16

_MESH = plsc.VectorSubcoreMesh(core_axis_name="c", subcore_axis_name="s")


def _zero_vmem(ref, nrows, d):
    zv = jnp.zeros((16,), jnp.float32)

    def row(r, _):
        def col(j, _):
            ref[r, pl.ds(j * 16, 16)] = zv
            return 0

        return lax.fori_loop(0, d // 16, col, 0)

    lax.fori_loop(0, nrows, row, 0)


NCH = EW // CH


def _make_sc_aggregate(d):
    rps = NP // NS
    seg1 = 64
    seg2 = NCH - seg1

    @functools.partial(
        pl.kernel,
        out_type=jax.ShapeDtypeStruct((NC, NP, d), jnp.float32),
        mesh=_MESH,
        scratch_types=[
            pltpu.VMEM((seg1, CH), jnp.int32),
            pltpu.VMEM((seg1, CH), jnp.int32),
            pltpu.VMEM((CH, d), jnp.float32),
            pltpu.VMEM((CH, d), jnp.float32),
            pltpu.VMEM_SHARED((NP, d), jnp.float32),
            pltpu.SemaphoreType.DMA,
            pltpu.SemaphoreType.DMA,
        ],
    )
    def agg(h_hbm, src_hbm, dst_hbm, out_hbm,
            src_v, dst_v, buf0, buf1, acc_sh, sem0, sem1):
        c = lax.axis_index("c")
        s = lax.axis_index("s")
        w = c * NS + s

        _zero_vmem(buf0, ZR, d)
        for i in range(rps // ZR):
            pltpu.sync_copy(buf0.at[pl.ds(0, ZR)], acc_sh.at[pl.ds(s * rps + i * ZR, ZR)])
        plsc.subcore_barrier()

        def segment(base, count):
            pltpu.sync_copy(src_hbm.at[w, pl.ds(base, count)],
                            src_v.at[pl.ds(0, count)])
            pltpu.sync_copy(dst_hbm.at[w, pl.ds(base, count)],
                            dst_v.at[pl.ds(0, count)])
            pltpu.async_copy(h_hbm.at[src_v.at[0]], buf0, sem0)

            def pair(p, _):
                m0 = p * 2
                m1 = m0 + 1
                pltpu.make_async_copy(h_hbm.at[src_v.at[m0]], buf0, sem0).wait()
                pltpu.async_copy(h_hbm.at[src_v.at[m1]], buf1, sem1)
                pltpu.sync_copy(buf0, acc_sh.at[dst_v.at[m0]], add=True)
                pltpu.make_async_copy(h_hbm.at[src_v.at[m1]], buf1, sem1).wait()
                pltpu.async_copy(h_hbm.at[src_v.at[m1 + 1]], buf0, sem0)
                pltpu.sync_copy(buf1, acc_sh.at[dst_v.at[m1]], add=True)
                return 0

            if count % 2:
                lax.fori_loop(0, (count - 1) // 2, pair, 0)
                pltpu.make_async_copy(h_hbm.at[src_v.at[count - 1]], buf0, sem0).wait()
                pltpu.sync_copy(buf0, acc_sh.at[dst_v.at[count - 1]], add=True)
            else:
                lax.fori_loop(0, (count - 2) // 2, pair, 0)
                pltpu.make_async_copy(h_hbm.at[src_v.at[count - 2]], buf0, sem0).wait()
                pltpu.async_copy(h_hbm.at[src_v.at[count - 1]], buf1, sem1)
                pltpu.sync_copy(buf0, acc_sh.at[dst_v.at[count - 2]], add=True)
                pltpu.make_async_copy(h_hbm.at[src_v.at[count - 1]], buf1, sem1).wait()
                pltpu.sync_copy(buf1, acc_sh.at[dst_v.at[count - 1]], add=True)

        segment(0, seg1)
        segment(seg1, seg2)
        plsc.subcore_barrier()

        pltpu.sync_copy(
            acc_sh.at[pl.ds(s * rps, rps)],
            out_hbm.at[c, pl.ds(s * rps, rps)],
        )

    return agg


_sc_aggregate_128 = _make_sc_aggregate(128)


_RPS = NP // NS


@functools.partial(
    pl.kernel,
    out_type=jax.ShapeDtypeStruct((NC, NP, 128), jnp.float32),
    mesh=_MESH,
    scratch_types=[
        pltpu.VMEM((IB, CH), jnp.int32),
        pltpu.VMEM((CH, 128), jnp.float32),
        pltpu.VMEM((ZR, 128), jnp.float32),
        pltpu.VMEM_SHARED((NP, 128), jnp.float32),
    ],
)
def _sc_histogram(ones_hbm, idx_hbm, out_hbm, idx_v, ones_v, z_v, acc_sh):
    c = lax.axis_index("c")
    s = lax.axis_index("s")
    w = c * NS + s

    _zero_vmem(z_v, ZR, 128)
    for i in range(_RPS // ZR):
        pltpu.sync_copy(z_v, acc_sh.at[pl.ds(s * _RPS + i * ZR, ZR)])
    plsc.subcore_barrier()

    for k in range(4):
        pltpu.sync_copy(ones_hbm.at[k], ones_v)

        def block(b, _, _k=k):
            pltpu.sync_copy(idx_hbm.at[w, _k, b], idx_v)

            def chunk(j, _):
                pltpu.sync_copy(ones_v, acc_sh.at[idx_v.at[j]], add=True)
                return 0

            lax.fori_loop(0, IB, chunk, 0)
            return 0

        lax.fori_loop(0, NB, block, 0)
    plsc.subcore_barrier()

    pltpu.sync_copy(
        acc_sh.at[pl.ds(s * _RPS, _RPS)],
        out_hbm.at[c, pl.ds(s * _RPS, _RPS)],
    )


def _norm_col(degp_ref, k):
    deg = degp_ref[0, :, k:k + 1] + degp_ref[1, :, k:k + 1]
    return lax.rsqrt(jnp.maximum(deg, 1.0))


def _scale_body(x_ref, degp_ref, o_ref):
    o_ref[...] = x_ref[...] * _norm_col(degp_ref, 0)


def _mm_body(aggp_ref, deg_ref, w1_ref, b1_ref, w2_ref, o_ref):
    a = (aggp_ref[0] + aggp_ref[1]) * _norm_col(deg_ref, 1)
    m = jnp.dot(a, w1_ref[...], preferred_element_type=jnp.float32) + b1_ref[...]
    m = m * _norm_col(deg_ref, 2)
    o_ref[...] = jnp.dot(m, w2_ref[...], preferred_element_type=jnp.float32)


def _final_body(aggp_ref, deg_ref, b2_ref, o_ref):
    a = (aggp_ref[0, :, 0:64] + aggp_ref[1, :, 0:64]) * _norm_col(deg_ref, 3)
    o_ref[...] = jax.nn.sigmoid(a + b2_ref[...])


_BR = 2000
_tc_scale = pl.pallas_call(
    _scale_body,
    grid=(N // _BR,),
    in_specs=[
        pl.BlockSpec((_BR, 128), lambda i: (i, 0)),
        pl.BlockSpec((NC, _BR, HW), lambda i: (0, i, 0)),
    ],
    out_specs=pl.BlockSpec((_BR, 128), lambda i: (i, 0)),
    out_shape=jax.ShapeDtypeStruct((N, 128), jnp.float32),
)

_BM = 1000
_tc_matmul = pl.pallas_call(
    _mm_body,
    grid=(N // _BM,),
    in_specs=[
        pl.BlockSpec((NC, _BM, 128), lambda i: (0, i, 0)),
        pl.BlockSpec((NC, _BM, HW), lambda i: (0, i, 0)),
        pl.BlockSpec((128, 1000), lambda i: (0, 0)),
        pl.BlockSpec((1, 1000), lambda i: (0, 0)),
        pl.BlockSpec((1000, 128), lambda i: (0, 0)),
    ],
    out_specs=pl.BlockSpec((_BM, 128), lambda i: (i, 0)),
    out_shape=jax.ShapeDtypeStruct((N, 128), jnp.float32),
)

_tc_final = pl.pallas_call(
    _final_body,
    grid=(N // _BR,),
    in_specs=[
        pl.BlockSpec((NC, _BR, 128), lambda i: (0, i, 0)),
        pl.BlockSpec((NC, _BR, HW), lambda i: (0, i, 0)),
        pl.BlockSpec((1, 64), lambda i: (0, 0)),
    ],
    out_specs=pl.BlockSpec((_BR, 64), lambda i: (i, 0)),
    out_shape=jax.ShapeDtypeStruct((N, 64), jnp.float32),
)


def kernel(x, edge_index1, edge_index2, W1, b1, W2, b2):
    src1 = edge_index1[0].astype(jnp.int32).reshape(NW, NB, IB, CH)
    dst1 = edge_index1[1].astype(jnp.int32).reshape(NW, NB, IB, CH)
    src2 = edge_index2[0].astype(jnp.int32).reshape(NW, NB, IB, CH)
    dst2 = edge_index2[1].astype(jnp.int32).reshape(NW, NB, IB, CH)
    asrc1 = src1.reshape(NW, NCH, CH)
    adst1 = dst1.reshape(NW, NCH, CH)
    asrc2 = src2.reshape(NW, NCH, CH)
    adst2 = dst2.reshape(NW, NCH, CH)

    idxs = jnp.stack([src1, dst1, src2, dst2], axis=1)
    onehots = jnp.broadcast_to(
        jnp.eye(128, dtype=jnp.float32)[:4][:, None, :], (4, CH, 128)
    )
    hist = _sc_histogram(onehots, idxs)[:, :, :HW]

    h1 = _tc_scale(x, hist)
    agg1 = _sc_aggregate_128(h1, asrc1, adst1)
    W2p = jnp.pad(W2, ((0, 0), (0, 64)))
    t = _tc_matmul(agg1, hist, W1, b1.reshape(1, 1000), W2p)
    agg2 = _sc_aggregate_128(t, asrc2, adst2)
    return _tc_final(agg2, hist, b2.reshape(1, 64))


# speedup vs baseline: 13.8732x; 1.1115x over previous
import functools

import jax
import jax.numpy as jnp
from jax import lax
from jax.experimental import pallas as pl
from jax.experimental.pallas import tpu as pltpu
from jax.experimental.pallas import tpu_sc as plsc

N = 10000
NP = 10240
E = 320000
NC, NS = 2, 16
NW = NC * NS
EW = E // NW
CH = 80
IB = 25
NB = EW // (IB * CH)
ZR = 32
HW = 16

_MESH = plsc.VectorSubcoreMesh(core_axis_name="c", subcore_axis_name="s")


def _zero_vmem(ref, nrows, d):
    zv = jnp.zeros((16,), jnp.float32)

    def row(r, _):
        def col(j, _):
            ref[r, pl.ds(j * 16, 16)] = zv
            return 0

        return lax.fori_loop(0, d // 16, col, 0)

    lax.fori_loop(0, nrows, row, 0)


NCH = EW // CH


def _make_sc_aggregate(d):
    rps = NP // NS
    seg1 = 64
    seg2 = NCH - seg1

    @functools.partial(
        pl.kernel,
        out_type=jax.ShapeDtypeStruct((NC, NP, d), jnp.float32),
        mesh=_MESH,
        scratch_types=[
            pltpu.VMEM((seg1, CH), jnp.int32),
            pltpu.VMEM((seg1, CH), jnp.int32),
            pltpu.VMEM((CH, d), jnp.float32),
            pltpu.VMEM((CH, d), jnp.float32),
            pltpu.VMEM_SHARED((NP, d), jnp.float32),
            pltpu.SemaphoreType.DMA,
            pltpu.SemaphoreType.DMA,
        ],
    )
    def agg(h_hbm, src_hbm, dst_hbm, out_hbm,
            src_v, dst_v, buf0, buf1, acc_sh, sem0, sem1):
        c = lax.axis_index("c")
        s = lax.axis_index("s")
        w = c * NS + s

        _zero_vmem(buf0, ZR, d)
        for i in range(rps // ZR):
            pltpu.sync_copy(buf0.at[pl.ds(0, ZR)], acc_sh.at[pl.ds(s * rps + i * ZR, ZR)])
        plsc.subcore_barrier()

        def segment(base, count):
            pltpu.sync_copy(src_hbm.at[w, pl.ds(base, count)],
                            src_v.at[pl.ds(0, count)])
            pltpu.sync_copy(dst_hbm.at[w, pl.ds(base, count)],
                            dst_v.at[pl.ds(0, count)])
            pltpu.async_copy(h_hbm.at[src_v.at[0]], buf0, sem0)

            def pair(p, _):
                m0 = p * 2
                m1 = m0 + 1
                pltpu.make_async_copy(h_hbm.at[src_v.at[m0]], buf0, sem0).wait()
                pltpu.async_copy(h_hbm.at[src_v.at[m1]], buf1, sem1)
                pltpu.sync_copy(buf0, acc_sh.at[dst_v.at[m0]], add=True)
                pltpu.make_async_copy(h_hbm.at[src_v.at[m1]], buf1, sem1).wait()
                pltpu.async_copy(h_hbm.at[src_v.at[m1 + 1]], buf0, sem0)
                pltpu.sync_copy(buf1, acc_sh.at[dst_v.at[m1]], add=True)
                return 0

            if count % 2:
                lax.fori_loop(0, (count - 1) // 2, pair, 0)
                pltpu.make_async_copy(h_hbm.at[src_v.at[count - 1]], buf0, sem0).wait()
                pltpu.sync_copy(buf0, acc_sh.at[dst_v.at[count - 1]], add=True)
            else:
                lax.fori_loop(0, (count - 2) // 2, pair, 0)
                pltpu.make_async_copy(h_hbm.at[src_v.at[count - 2]], buf0, sem0).wait()
                pltpu.async_copy(h_hbm.at[src_v.at[count - 1]], buf1, sem1)
                pltpu.sync_copy(buf0, acc_sh.at[dst_v.at[count - 2]], add=True)
                pltpu.make_async_copy(h_hbm.at[src_v.at[count - 1]], buf1, sem1).wait()
                pltpu.sync_copy(buf1, acc_sh.at[dst_v.at[count - 1]], add=True)

        segment(0, seg1)
        segment(seg1, seg2)
        plsc.subcore_barrier()

        pltpu.sync_copy(
            acc_sh.at[pl.ds(s * rps, rps)],
            out_hbm.at[c, pl.ds(s * rps, rps)],
        )

    return agg


_sc_aggregate_128 = _make_sc_aggregate(128)


_RPS = NP // NS


@functools.partial(
    pl.kernel,
    out_type=jax.ShapeDtypeStruct((NC, NP, 128), jnp.float32),
    mesh=_MESH,
    scratch_types=[
        pltpu.VMEM((IB, CH), jnp.int32),
        pltpu.VMEM((CH, 128), jnp.float32),
        pltpu.VMEM((ZR, 128), jnp.float32),
        pltpu.VMEM_SHARED((NP, 128), jnp.float32),
    ],
)
def _sc_histogram(ones_hbm, idx_hbm, out_hbm, idx_v, ones_v, z_v, acc_sh):
    c = lax.axis_index("c")
    s = lax.axis_index("s")
    w = c * NS + s

    _zero_vmem(z_v, ZR, 128)
    for i in range(_RPS // ZR):
        pltpu.sync_copy(z_v, acc_sh.at[pl.ds(s * _RPS + i * ZR, ZR)])
    plsc.subcore_barrier()

    for k in range(3):
        pltpu.sync_copy(ones_hbm.at[k], ones_v)

        def block(b, _, _k=k):
            pltpu.sync_copy(idx_hbm.at[w, _k, b], idx_v)

            def chunk(j, _):
                pltpu.sync_copy(ones_v, acc_sh.at[idx_v.at[j]], add=True)
                return 0

            lax.fori_loop(0, IB, chunk, 0)
            return 0

        lax.fori_loop(0, NB, block, 0)
    plsc.subcore_barrier()

    pltpu.sync_copy(
        acc_sh.at[pl.ds(s * _RPS, _RPS)],
        out_hbm.at[c, pl.ds(s * _RPS, _RPS)],
    )


def _norm_col(degp_ref, k):
    deg = degp_ref[0, :, k:k + 1] + degp_ref[1, :, k:k + 1]
    return lax.rsqrt(jnp.maximum(deg, 1.0))


def _scale_body(x_ref, degp_ref, o_ref):
    o_ref[...] = x_ref[...] * _norm_col(degp_ref, 0)


def _mm_body(aggp_ref, deg_ref, w1_ref, b1_ref, w2_ref, o_ref):
    a = (aggp_ref[0] + aggp_ref[1]) * _norm_col(deg_ref, 1)
    m = jnp.dot(a, w1_ref[...], preferred_element_type=jnp.float32) + b1_ref[...]
    m = m * _norm_col(deg_ref, 2)
    t = jnp.dot(m, w2_ref[...], preferred_element_type=jnp.float32)
    col = lax.broadcasted_iota(jnp.int32, t.shape, 1)
    o_ref[...] = t + (col == 64).astype(jnp.float32)


def _final_body(aggp_ref, b2_ref, o_ref):
    deg2 = aggp_ref[0, :, 64:65] + aggp_ref[1, :, 64:65]
    norm = lax.rsqrt(jnp.maximum(deg2, 1.0))
    a = (aggp_ref[0, :, 0:64] + aggp_ref[1, :, 0:64]) * norm
    o_ref[...] = jax.nn.sigmoid(a + b2_ref[...])


_BR = 2000
_tc_scale = pl.pallas_call(
    _scale_body,
    grid=(N // _BR,),
    in_specs=[
        pl.BlockSpec((_BR, 128), lambda i: (i, 0)),
        pl.BlockSpec((NC, _BR, HW), lambda i: (0, i, 0)),
    ],
    out_specs=pl.BlockSpec((_BR, 128), lambda i: (i, 0)),
    out_shape=jax.ShapeDtypeStruct((N, 128), jnp.float32),
)

_BM = 1000
_tc_matmul = pl.pallas_call(
    _mm_body,
    grid=(N // _BM,),
    in_specs=[
        pl.BlockSpec((NC, _BM, 128), lambda i: (0, i, 0)),
        pl.BlockSpec((NC, _BM, HW), lambda i: (0, i, 0)),
        pl.BlockSpec((128, 1000), lambda i: (0, 0)),
        pl.BlockSpec((1, 1000), lambda i: (0, 0)),
        pl.BlockSpec((1000, 128), lambda i: (0, 0)),
    ],
    out_specs=pl.BlockSpec((_BM, 128), lambda i: (i, 0)),
    out_shape=jax.ShapeDtypeStruct((N, 128), jnp.float32),
)

_tc_final = pl.pallas_call(
    _final_body,
    grid=(N // _BR,),
    in_specs=[
        pl.BlockSpec((NC, _BR, 128), lambda i: (0, i, 0)),
        pl.BlockSpec((1, 64), lambda i: (0, 0)),
    ],
    out_specs=pl.BlockSpec((_BR, 64), lambda i: (i, 0)),
    out_shape=jax.ShapeDtypeStruct((N, 64), jnp.float32),
)


def kernel(x, edge_index1, edge_index2, W1, b1, W2, b2):
    src1 = edge_index1[0].astype(jnp.int32).reshape(NW, NB, IB, CH)
    dst1 = edge_index1[1].astype(jnp.int32).reshape(NW, NB, IB, CH)
    src2 = edge_index2[0].astype(jnp.int32).reshape(NW, NB, IB, CH)
    dst2 = edge_index2[1].astype(jnp.int32).reshape(NW, NB, IB, CH)
    asrc1 = src1.reshape(NW, NCH, CH)
    adst1 = dst1.reshape(NW, NCH, CH)
    asrc2 = src2.reshape(NW, NCH, CH)
    adst2 = dst2.reshape(NW, NCH, CH)

    idxs = jnp.stack([src1, dst1, src2], axis=1)
    onehots = jnp.broadcast_to(
        jnp.eye(128, dtype=jnp.float32)[:3][:, None, :], (3, CH, 128)
    )
    hist = _sc_histogram(onehots, idxs)[:, :, :HW]

    h1 = _tc_scale(x, hist)
    agg1 = _sc_aggregate_128(h1, asrc1, adst1)
    W2p = jnp.pad(W2, ((0, 0), (0, 64)))
    t = _tc_matmul(agg1, hist, W1, b1.reshape(1, 1000), W2p)
    agg2 = _sc_aggregate_128(t, asrc2, adst2)
    return _tc_final(agg2, b2.reshape(1, 64))
